# baseline (speedup 1.0000x reference)
import functools
import math

import jax
import jax.numpy as jnp
from jax import lax
from jax.experimental import pallas as pl
from jax.experimental.pallas import tpu as pltpu
from jax.experimental.pallas import tpu_sc as plsc

VOCAB_SIZE = 1000000
D_MODEL = 64
SCALE = math.sqrt(D_MODEL)

NC = 2
NS = 16
NW = NC * NS
LANES = 16

B_ROWS = 4096 * 200
BPW = B_ROWS // NW
CHUNK = 128
NCH = BPW // CHUNK


def _emb_body(idx_hbm, table_hbm, out_hbm, idx_v, rows_v, sem_g, sem_i):
    wid = lax.axis_index("s") * NC + lax.axis_index("c")
    base = wid * BPW

    pltpu.async_copy(idx_hbm.at[wid], idx_v, sem_i).wait()

    def chunk_body(c, _):
        pltpu.async_copy(table_hbm.at[idx_v.at[c]], rows_v, sem_g).wait()

        def row_body(i, _):
            for j in range(D_MODEL // LANES):
                sl = pl.ds(j * LANES, LANES)
                rows_v[i, sl] = rows_v[i, sl] * SCALE
            return 0

        lax.fori_loop(0, CHUNK, row_body, 0, unroll=2)

        pltpu.sync_copy(rows_v, out_hbm.at[pl.ds(base + c * CHUNK, CHUNK)])
        return 0

    lax.fori_loop(0, NCH, chunk_body, 0)


@jax.jit
def _emb(x_grouped, table):
    mesh = plsc.VectorSubcoreMesh(
        core_axis_name="c", subcore_axis_name="s", num_cores=NC,
        num_subcores=NS)
    f = functools.partial(
        pl.kernel,
        out_type=jax.ShapeDtypeStruct((B_ROWS, D_MODEL), jnp.float32),
        mesh=mesh,
        scratch_types=[
            pltpu.VMEM((NCH, CHUNK), jnp.int32),
            pltpu.VMEM((CHUNK, D_MODEL), jnp.float32),
            pltpu.SemaphoreType.DMA,
            pltpu.SemaphoreType.DMA,
        ],
        compiler_params=pltpu.CompilerParams(use_tc_tiling_on_sc=False),
    )(_emb_body)
    return f(x_grouped, table)


def kernel(x, table):
    x_grouped = x.reshape(NW, NCH, CHUNK).astype(jnp.int32)
    out = _emb(x_grouped, table)
    return out.reshape(x.shape[0], x.shape[1], D_MODEL)


# speedup vs baseline: 1.0562x; 1.0562x over previous
import functools
import math

import jax
import jax.numpy as jnp
from jax import lax
from jax.experimental import pallas as pl
from jax.experimental.pallas import tpu as pltpu
from jax.experimental.pallas import tpu_sc as plsc

VOCAB_SIZE = 1000000
D_MODEL = 64
SCALE = math.sqrt(D_MODEL)

NC = 2
NS = 16
NW = NC * NS
LANES = 16

B_ROWS = 4096 * 200
BPW = B_ROWS // NW
CHUNK = 128
NCH = BPW // CHUNK
K = 4
NG = NCH // K


def _emb_body(idx_hbm, table_hbm, out_hbm, idx_v, in_v, out_v, *sems):
    gsem = sems[:K]
    ssem = sems[K:2 * K]
    sem_i = sems[2 * K]
    wid = lax.axis_index("s") * NC + lax.axis_index("c")
    base = wid * BPW

    pltpu.async_copy(idx_hbm.at[wid], idx_v, sem_i).wait()

    def start_gather(b, c):
        pltpu.make_async_copy(table_hbm.at[idx_v.at[c]], in_v.at[b],
                              gsem[b]).start()

    def wait_gather(b, c):
        pltpu.make_async_copy(table_hbm.at[idx_v.at[c]], in_v.at[b],
                              gsem[b]).wait()

    def start_store(b, c):
        pltpu.make_async_copy(out_v.at[b],
                              out_hbm.at[pl.ds(base + c * CHUNK, CHUNK)],
                              ssem[b]).start()

    def wait_store(b, c):
        pltpu.make_async_copy(out_v.at[b],
                              out_hbm.at[pl.ds(base + c * CHUNK, CHUNK)],
                              ssem[b]).wait()

    def scale_rows(b):
        src = in_v.at[b]
        dst = out_v.at[b]

        def row_body(i, _):
            for j in range(D_MODEL // LANES):
                sl = pl.ds(j * LANES, LANES)
                dst[i, sl] = src[i, sl] * SCALE
            return 0

        lax.fori_loop(0, CHUNK, row_body, 0, unroll=4)

    for b in range(K):
        start_gather(b, b)

    def group_body(g, _):
        for b in range(K):
            c = g * K + b
            wait_gather(b, c)

            @pl.when(g > 0)
            def _():
                wait_store(b, c - K)

            scale_rows(b)
            start_store(b, c)
            start_gather(b, c + K)
        return 0

    lax.fori_loop(0, NG - 1, group_body, 0)

    for b in range(K):
        c = (NG - 1) * K + b
        wait_gather(b, c)
        wait_store(b, c - K)
        scale_rows(b)
        start_store(b, c)
    for b in range(K):
        wait_store(b, (NG - 1) * K + b)


@jax.jit
def _emb(x_grouped, table):
    mesh = plsc.VectorSubcoreMesh(
        core_axis_name="c", subcore_axis_name="s", num_cores=NC,
        num_subcores=NS)
    f = functools.partial(
        pl.kernel,
        out_type=jax.ShapeDtypeStruct((B_ROWS, D_MODEL), jnp.float32),
        mesh=mesh,
        scratch_types=[
            pltpu.VMEM((NCH, CHUNK), jnp.int32),
            pltpu.VMEM((K, CHUNK, D_MODEL), jnp.float32),
            pltpu.VMEM((K, CHUNK, D_MODEL), jnp.float32),
        ] + [pltpu.SemaphoreType.DMA] * (2 * K + 1),
        compiler_params=pltpu.CompilerParams(use_tc_tiling_on_sc=False),
    )(_emb_body)
    return f(x_grouped, table)


def kernel(x, table):
    x_grouped = x.reshape(NW, NCH, CHUNK).astype(jnp.int32)
    out = _emb(x_grouped, table)
    return out.reshape(x.shape[0], x.shape[1], D_MODEL)


# speedup vs baseline: 1.0566x; 1.0004x over previous
import functools
import math

import jax
import jax.numpy as jnp
from jax import lax
from jax.experimental import pallas as pl
from jax.experimental.pallas import tpu as pltpu
from jax.experimental.pallas import tpu_sc as plsc

VOCAB_SIZE = 1000000
D_MODEL = 64
SCALE = math.sqrt(D_MODEL)

NC = 2
NS = 16
NW = NC * NS
LANES = 16

B_ROWS = 4096 * 200
BPW = B_ROWS // NW
CHUNK = 128
NCH = BPW // CHUNK
K = 4
NG = NCH // K


def _emb_body(idx_hbm, table_hbm, out_hbm, idx_v, in_v, out_v, *sems):
    gsem = sems[:K]
    ssem = sems[K:2 * K]
    sem_i = sems[2 * K]
    wid = lax.axis_index("s") * NC + lax.axis_index("c")
    base = wid * BPW

    pltpu.async_copy(idx_hbm.at[wid], idx_v, sem_i).wait()

    def start_gather(b, c):
        pltpu.make_async_copy(table_hbm.at[idx_v.at[c]], in_v.at[b],
                              gsem[b]).start()

    def wait_gather(b, c):
        pltpu.make_async_copy(table_hbm.at[idx_v.at[c]], in_v.at[b],
                              gsem[b]).wait()

    def start_store(b, c):
        pltpu.make_async_copy(out_v.at[b],
                              out_hbm.at[pl.ds(base + c * CHUNK, CHUNK)],
                              ssem[b]).start()

    def wait_store(b, c):
        pltpu.make_async_copy(out_v.at[b],
                              out_hbm.at[pl.ds(base + c * CHUNK, CHUNK)],
                              ssem[b]).wait()

    def scale_rows(b):
        src = in_v.at[b]
        dst = out_v.at[b]

        def row_body(i, _):
            for j in range(D_MODEL // LANES):
                sl = pl.ds(j * LANES, LANES)
                dst[i, sl] = src[i, sl] * SCALE
            return 0

        lax.fori_loop(0, CHUNK, row_body, 0, unroll=8)

    for b in range(K):
        start_gather(b, b)

    def group_body(g, _):
        for b in range(K):
            c = g * K + b
            wait_gather(b, c)

            @pl.when(g > 0)
            def _():
                wait_store(b, c - K)

            scale_rows(b)
            start_store(b, c)
            start_gather(b, c + K)
        return 0

    lax.fori_loop(0, NG - 1, group_body, 0)

    for b in range(K):
        c = (NG - 1) * K + b
        wait_gather(b, c)
        wait_store(b, c - K)
        scale_rows(b)
        start_store(b, c)
    for b in range(K):
        wait_store(b, (NG - 1) * K + b)


@jax.jit
def _emb(x_grouped, table):
    mesh = plsc.VectorSubcoreMesh(
        core_axis_name="c", subcore_axis_name="s", num_cores=NC,
        num_subcores=NS)
    f = functools.partial(
        pl.kernel,
        out_type=jax.ShapeDtypeStruct((B_ROWS, D_MODEL), jnp.float32),
        mesh=mesh,
        scratch_types=[
            pltpu.VMEM((NCH, CHUNK), jnp.int32),
            pltpu.VMEM((K, CHUNK, D_MODEL), jnp.float32),
            pltpu.VMEM((K, CHUNK, D_MODEL), jnp.float32),
        ] + [pltpu.SemaphoreType.DMA] * (2 * K + 1),
        compiler_params=pltpu.CompilerParams(use_tc_tiling_on_sc=False),
    )(_emb_body)
    return f(x_grouped, table)


def kernel(x, table):
    x_grouped = x.astype(jnp.int32).reshape(NW, NCH, CHUNK)
    out = _emb(x_grouped, table)
    return out.reshape(x.shape[0], x.shape[1], D_MODEL)
